# Initial kernel scaffold; baseline (speedup 1.0000x reference)
import jax
import jax.numpy as jnp
from jax.experimental import pallas as pl


def kernel(feat, edge_index, etypes, W1, b1, W2, b2, type_bias, centers, sigmas, a_r, b_r):
    raise NotImplementedError("write your pallas kernel here")


# speedup vs baseline: 4.7461x; 4.7461x over previous
import functools
import jax
import jax.numpy as jnp
from jax import lax
from jax.experimental import pallas as pl
from jax.experimental.pallas import tpu as pltpu
from jax.experimental.pallas import tpu_sc as plsc

F32 = jnp.float32
I32 = jnp.int32


def _sc_degrees(src, dst, n):
    e = src.shape[0]
    nw = 32
    epw = e // nw
    mesh = plsc.VectorSubcoreMesh(core_axis_name="c", subcore_axis_name="s")

    @functools.partial(
        pl.kernel, mesh=mesh,
        compiler_params=pltpu.CompilerParams(needs_layout_passes=False),
        out_type=jax.ShapeDtypeStruct((2 * nw * n,), F32),
        scratch_types=[
            pltpu.VMEM((epw,), I32),
            pltpu.VMEM((epw,), I32),
            pltpu.VMEM((n,), F32),
            pltpu.VMEM((n,), F32),
        ],
    )
    def k(src_h, dst_h, out_h, srcv, dstv, ho, hi):
        cid = lax.axis_index("c")
        sid = lax.axis_index("s")
        wid = cid * 16 + sid
        base = wid * epw
        pltpu.sync_copy(src_h.at[pl.ds(base, epw)], srcv)
        pltpu.sync_copy(dst_h.at[pl.ds(base, epw)], dstv)
        z16 = jnp.zeros((16,), F32)

        def zb(i, c):
            ho[pl.ds(i * 16, 16)] = z16
            hi[pl.ds(i * 16, 16)] = z16
            return c

        lax.fori_loop(0, n // 16, zb, 0)
        ones = jnp.ones((16,), F32)

        def hb(g, c):
            iv = srcv[pl.ds(g * 16, 16)]
            plsc.addupdate_scatter(ho, [iv], ones)
            jv = dstv[pl.ds(g * 16, 16)]
            plsc.addupdate_scatter(hi, [jv], ones)
            return c

        lax.fori_loop(0, epw // 16, hb, 0)
        pltpu.sync_copy(ho, out_h.at[pl.ds(wid * n, n)])
        pltpu.sync_copy(hi, out_h.at[pl.ds((nw + wid) * n, n)])

    return k(src, dst).reshape(2, nw, n)


def _tc_norms(degpart):
    _, nw, n = degpart.shape

    def body(dp_ref, out_ref):
        deg = jnp.sum(dp_ref[...], axis=1)
        out_ref[...] = lax.rsqrt(jnp.maximum(deg, 1.0))

    return pl.pallas_call(
        body,
        out_shape=jax.ShapeDtypeStruct((2, n), F32),
    )(degpart)


def _sc_edge1(feat, src, dst, etypes, norm_src, params):
    n, d = feat.shape
    e = src.shape[0]
    nw = 32
    epw = e // nw
    bsz = 80
    nb = epw // bsz
    ng = bsz // 16
    rows_bt = ((n // 16) // 16) * 16
    nch_base = rows_bt // 16
    nch_last = (n - 15 * rows_bt) // 16
    mesh = plsc.VectorSubcoreMesh(core_axis_name="c", subcore_axis_name="s")
    inv_sqrt_d = float(1.0 / (d ** 0.5))

    @functools.partial(
        pl.kernel, mesh=mesh,
        compiler_params=pltpu.CompilerParams(needs_layout_passes=False),
        out_type=(jax.ShapeDtypeStruct((e,), F32),
                  jax.ShapeDtypeStruct((2, n, d), F32)),
        scratch_types=[
            pltpu.VMEM((bsz, d), F32),
            pltpu.VMEM((bsz, d), F32),
            pltpu.VMEM((n,), F32),
            pltpu.VMEM((8, 16), F32),
            pltpu.VMEM((bsz,), I32),
            pltpu.VMEM((bsz,), I32),
            pltpu.VMEM((bsz,), I32),
            pltpu.VMEM((bsz,), F32),
            pltpu.VMEM((16, d), F32),
            pltpu.VMEM_SHARED((n, d), F32),
            pltpu.SemaphoreType.DMA,
            pltpu.SemaphoreType.DMA,
        ],
    )
    def k(feat_h, src_h, dst_h, et_h, norm_h, par_h, att_h, s_h,
          rows_a, rows_b, normv, parv, srcb, dstb, etb, attb,
          zbuf, s_sh, sem_a, sem_b):
        cid = lax.axis_index("c")
        sid = lax.axis_index("s")
        wid = cid * 16 + sid
        ebase = wid * epw
        pltpu.sync_copy(norm_h, normv)
        pltpu.sync_copy(par_h, parv)
        rstart = sid * rows_bt
        nch = jnp.where(sid == 15, nch_last, nch_base)
        z16 = jnp.zeros((16,), F32)

        def zb(i, c):
            for j in range(8):
                zbuf[i, pl.ds(j * 16, 16)] = z16
            return c

        lax.fori_loop(0, 16, zb, 0)

        def zs(i, c):
            pltpu.sync_copy(zbuf, s_sh.at[pl.ds(rstart + i * 16, 16)])
            return c

        lax.fori_loop(0, nch, zs, 0)
        plsc.subcore_barrier()

        zeros_i = jnp.zeros((16,), I32)

        def group(g, c):
            go = g * 16
            lanes = lax.iota(I32, 16)
            svec = jnp.zeros((16,), F32)
            for ee in range(16):
                r0 = go + ee
                acc = rows_a[r0, pl.ds(0, 16)] * rows_b[r0, pl.ds(0, 16)]
                for ch in range(1, 8):
                    acc = acc + (rows_a[r0, pl.ds(ch * 16, 16)]
                                 * rows_b[r0, pl.ds(ch * 16, 16)])
                svec = jnp.where(lanes == ee, jnp.sum(acc), svec)
            etv = etb[pl.ds(go, 16)]
            tb = plsc.load_gather(parv, [zeros_i, etv])
            s = svec * inv_sqrt_d + tb
            prow_c = parv[1, pl.ds(0, 16)]
            prow_w = parv[2, pl.ds(0, 16)]
            prow_a = parv[3, pl.ds(0, 16)]
            prow_b = parv[4, pl.ds(0, 16)]
            msum = jnp.zeros((16,), F32)
            yacc = jnp.zeros((16,), F32)
            for r in range(4):
                cr = prow_c[r]
                iw = prow_w[r]
                ar = prow_a[r]
                br = prow_b[r]
                dd = s - cr
                m = jnp.exp(dd * dd * iw)
                msum = msum + m
                yacc = yacc + m * (ar * s + br)
            z = yacc / (msum + 1e-9)
            att = 1.0 / (1.0 + jnp.exp(-z))
            attb[pl.ds(go, 16)] = att
            srcv = srcb[pl.ds(go, 16)]
            nrm = plsc.load_gather(normv, [srcv])
            scv = att * nrm
            for ee in range(16):
                r0 = go + ee
                sc = scv[ee]
                for ch in range(8):
                    rows_a[r0, pl.ds(ch * 16, 16)] = (
                        rows_a[r0, pl.ds(ch * 16, 16)] * sc)
            return c

        def batch(bt, c):
            eb = ebase + bt * bsz
            pltpu.sync_copy(src_h.at[pl.ds(eb, bsz)], srcb)
            pltpu.sync_copy(dst_h.at[pl.ds(eb, bsz)], dstb)
            pltpu.sync_copy(et_h.at[pl.ds(eb, bsz)], etb)
            cp_a = pltpu.async_copy(feat_h.at[srcb], rows_a, sem_a)
            cp_b = pltpu.async_copy(feat_h.at[dstb], rows_b, sem_b)
            cp_a.wait()
            cp_b.wait()
            lax.fori_loop(0, ng, group, 0)
            pltpu.sync_copy(attb, att_h.at[pl.ds(eb, bsz)])
            pltpu.sync_copy(rows_a, s_sh.at[dstb], add=True)
            return c

        lax.fori_loop(0, nb, batch, 0)
        plsc.subcore_barrier()

        def dp(i, c):
            pltpu.sync_copy(s_sh.at[pl.ds(rstart + i * 16, 16)],
                            s_h.at[cid, pl.ds(rstart + i * 16, 16)])
            return c

        lax.fori_loop(0, nch, dp, 0)

    return k(feat, src, dst, etypes, norm_src, params)


def _sc_edge2(y, src, dst, att):
    n, d = y.shape
    e = src.shape[0]
    nw = 32
    epw = e // nw
    bsz = 80
    nb = epw // bsz
    ng = bsz // 16
    rows_bt = ((n // 16) // 16) * 16
    nch_base = rows_bt // 16
    nch_last = (n - 15 * rows_bt) // 16
    mesh = plsc.VectorSubcoreMesh(core_axis_name="c", subcore_axis_name="s")

    @functools.partial(
        pl.kernel, mesh=mesh,
        compiler_params=pltpu.CompilerParams(needs_layout_passes=False),
        out_type=jax.ShapeDtypeStruct((2, n, d), F32),
        scratch_types=[
            pltpu.VMEM((bsz, d), F32),
            pltpu.VMEM((bsz,), I32),
            pltpu.VMEM((bsz,), I32),
            pltpu.VMEM((bsz,), F32),
            pltpu.VMEM((16, d), F32),
            pltpu.VMEM_SHARED((n, d), F32),
            pltpu.SemaphoreType.DMA,
        ],
    )
    def k(y_h, src_h, dst_h, att_h, s_h,
          rows, srcb, dstb, attbv, zbuf, s_sh, sem):
        cid = lax.axis_index("c")
        sid = lax.axis_index("s")
        wid = cid * 16 + sid
        ebase = wid * epw
        rstart = sid * rows_bt
        nch = jnp.where(sid == 15, nch_last, nch_base)
        z16 = jnp.zeros((16,), F32)

        def zb(i, c):
            for j in range(8):
                zbuf[i, pl.ds(j * 16, 16)] = z16
            return c

        lax.fori_loop(0, 16, zb, 0)

        def zs(i, c):
            pltpu.sync_copy(zbuf, s_sh.at[pl.ds(rstart + i * 16, 16)])
            return c

        lax.fori_loop(0, nch, zs, 0)
        plsc.subcore_barrier()

        def group(g, c):
            go = g * 16
            av16 = attbv[pl.ds(go, 16)]
            for ee in range(16):
                r0 = go + ee
                av = av16[ee]
                for ch in range(8):
                    rows[r0, pl.ds(ch * 16, 16)] = (
                        rows[r0, pl.ds(ch * 16, 16)] * av)
            return c

        def batch(bt, c):
            eb = ebase + bt * bsz
            pltpu.sync_copy(src_h.at[pl.ds(eb, bsz)], srcb)
            pltpu.sync_copy(dst_h.at[pl.ds(eb, bsz)], dstb)
            pltpu.sync_copy(att_h.at[pl.ds(eb, bsz)], attbv)
            pltpu.async_copy(y_h.at[srcb], rows, sem).wait()
            lax.fori_loop(0, ng, group, 0)
            pltpu.sync_copy(rows, s_sh.at[dstb], add=True)
            return c

        lax.fori_loop(0, nb, batch, 0)
        plsc.subcore_barrier()

        def dp(i, c):
            pltpu.sync_copy(s_sh.at[pl.ds(rstart + i * 16, 16)],
                            s_h.at[cid, pl.ds(rstart + i * 16, 16)])
            return c

        lax.fori_loop(0, nch, dp, 0)

    return k(y, src, dst, att)


def _tc_layer1(s1, w, b_row, ns_col, nd_col):
    _, n, d = s1.shape
    h = w.shape[1]
    blk = 2000

    def body(s_ref, w_ref, b_ref, ns_ref, nd_ref, out_ref):
        s = s_ref[0] + s_ref[1]
        hh = jnp.dot(s, w_ref[...], preferred_element_type=F32)
        x = jnp.maximum(hh * nd_ref[...] + b_ref[...], 0.0)
        out_ref[...] = x * ns_ref[...]

    return pl.pallas_call(
        body,
        grid=(n // blk,),
        in_specs=[
            pl.BlockSpec((2, blk, d), lambda i: (0, i, 0)),
            pl.BlockSpec((d, h), lambda i: (0, 0)),
            pl.BlockSpec((1, h), lambda i: (0, 0)),
            pl.BlockSpec((blk, 1), lambda i: (i, 0)),
            pl.BlockSpec((blk, 1), lambda i: (i, 0)),
        ],
        out_specs=pl.BlockSpec((blk, h), lambda i: (i, 0)),
        out_shape=jax.ShapeDtypeStruct((n, h), F32),
    )(s1, w, b_row, ns_col, nd_col)


def _tc_final(s2, w, b_row, nd_col):
    _, n, d = s2.shape
    h = w.shape[1]
    blk = 2000
    nblk = n // blk
    neg = -3.0e38

    def body(s_ref, w_ref, b_ref, nd_ref, out_ref):
        k = pl.program_id(0)
        s = s_ref[0] + s_ref[1]
        hh = jnp.dot(s, w_ref[...], preferred_element_type=F32)
        x = jnp.maximum(hh * nd_ref[...] + b_ref[...], 0.0)
        rows = (k * blk
                + lax.broadcasted_iota(I32, (blk, h), 0))
        valid = rows >= 2
        psum = jnp.sum(jnp.where(valid, x, 0.0), axis=0, keepdims=True)
        pmax = jnp.max(jnp.where(valid, x, neg), axis=0, keepdims=True)
        pmin = jnp.min(jnp.where(valid, x, -neg), axis=0, keepdims=True)

        @pl.when(k == 0)
        def _():
            out_ref[0:1, :] = x[0:1, :]
            out_ref[1:2, :] = x[1:2, :]
            out_ref[2:3, :] = psum
            out_ref[3:4, :] = pmax
            out_ref[4:5, :] = pmin

        @pl.when(k > 0)
        def _():
            out_ref[2:3, :] = out_ref[2:3, :] + psum
            out_ref[3:4, :] = jnp.maximum(out_ref[3:4, :], pmax)
            out_ref[4:5, :] = jnp.minimum(out_ref[4:5, :], pmin)

        @pl.when(k == nblk - 1)
        def _():
            out_ref[2:3, :] = out_ref[2:3, :] * (1.0 / (n - 2))

    return pl.pallas_call(
        body,
        grid=(nblk,),
        in_specs=[
            pl.BlockSpec((2, blk, d), lambda i: (0, i, 0)),
            pl.BlockSpec((d, h), lambda i: (0, 0)),
            pl.BlockSpec((1, h), lambda i: (0, 0)),
            pl.BlockSpec((blk, 1), lambda i: (i, 0)),
        ],
        out_specs=pl.BlockSpec((5, h), lambda i: (0, 0)),
        out_shape=jax.ShapeDtypeStruct((5, h), F32),
    )(s2, w, b_row, nd_col)


def kernel(feat, edge_index, etypes, W1, b1, W2, b2,
           type_bias, centers, sigmas, a_r, b_r):
    n = feat.shape[0]
    src = edge_index[0]
    dst = edge_index[1]
    params = jnp.zeros((8, 16), F32)
    params = (params.at[0, :4].set(type_bias)
                    .at[1, :4].set(centers)
                    .at[2, :4].set(-0.5 / (sigmas * sigmas))
                    .at[3, :4].set(a_r)
                    .at[4, :4].set(b_r))
    degpart = _sc_degrees(src, dst, n)
    norms = _tc_norms(degpart)
    ns_col = norms[0].reshape(n, 1)
    nd_col = norms[1].reshape(n, 1)
    att, s1 = _sc_edge1(feat, src, dst, etypes, norms[0], params)
    y = _tc_layer1(s1, W1, b1.reshape(1, -1), ns_col, nd_col)
    s2 = _sc_edge2(y, src, dst, att)
    return _tc_final(s2, W2, b2.reshape(1, -1), nd_col)


# speedup vs baseline: 8.4657x; 1.7837x over previous
import functools
import jax
import jax.numpy as jnp
from jax import lax
from jax.experimental import pallas as pl
from jax.experimental.pallas import tpu as pltpu
from jax.experimental.pallas import tpu_sc as plsc

F32 = jnp.float32
I32 = jnp.int32


def _sc_degrees(src, dst, n):
    e = src.shape[0]
    nw = 32
    epw = e // nw
    mesh = plsc.VectorSubcoreMesh(core_axis_name="c", subcore_axis_name="s")

    @functools.partial(
        pl.kernel, mesh=mesh,
        compiler_params=pltpu.CompilerParams(needs_layout_passes=False),
        out_type=jax.ShapeDtypeStruct((2 * nw * n,), F32),
        scratch_types=[
            pltpu.VMEM((epw,), I32),
            pltpu.VMEM((epw,), I32),
            pltpu.VMEM((n,), F32),
            pltpu.VMEM((n,), F32),
        ],
    )
    def k(src_h, dst_h, out_h, srcv, dstv, ho, hi):
        cid = lax.axis_index("c")
        sid = lax.axis_index("s")
        wid = cid * 16 + sid
        base = wid * epw
        pltpu.sync_copy(src_h.at[pl.ds(base, epw)], srcv)
        pltpu.sync_copy(dst_h.at[pl.ds(base, epw)], dstv)
        z16 = jnp.zeros((16,), F32)

        def zb(i, c):
            ho[pl.ds(i * 16, 16)] = z16
            hi[pl.ds(i * 16, 16)] = z16
            return c

        lax.fori_loop(0, n // 16, zb, 0)
        ones = jnp.ones((16,), F32)

        def hb(g, c):
            iv = srcv[pl.ds(g * 16, 16)]
            plsc.addupdate_scatter(ho, [iv], ones)
            jv = dstv[pl.ds(g * 16, 16)]
            plsc.addupdate_scatter(hi, [jv], ones)
            return c

        lax.fori_loop(0, epw // 16, hb, 0)
        pltpu.sync_copy(ho, out_h.at[pl.ds(wid * n, n)])
        pltpu.sync_copy(hi, out_h.at[pl.ds((nw + wid) * n, n)])

    return k(src, dst).reshape(2, nw, n)


def _tc_norms(degpart):
    _, nw, n = degpart.shape

    def body(dp_ref, out_ref):
        deg = jnp.sum(dp_ref[...], axis=1)
        out_ref[...] = lax.rsqrt(jnp.maximum(deg, 1.0))

    return pl.pallas_call(
        body,
        out_shape=jax.ShapeDtypeStruct((2, n), F32),
    )(degpart)


def _sc_edge1(feat, e3, normpk, params):
    n, d = feat.shape
    nw, nb, _, _, bsz = e3.shape
    ng = bsz // 16
    rows_bt = ((n // 16) // 16) * 16
    nch_base = rows_bt // 16
    nch_last = (n - 15 * rows_bt) // 16
    mesh = plsc.VectorSubcoreMesh(core_axis_name="c", subcore_axis_name="s")
    inv_sqrt_d = float(1.0 / (d ** 0.5))

    @functools.partial(
        pl.kernel, mesh=mesh,
        compiler_params=pltpu.CompilerParams(needs_layout_passes=False),
        out_type=(jax.ShapeDtypeStruct((nw, nb, 1, bsz), F32),
                  jax.ShapeDtypeStruct((2, n, d), F32)),
        scratch_types=[
            pltpu.VMEM((2, bsz, d), F32),
            pltpu.VMEM((2, bsz, d), F32),
            pltpu.VMEM((n // 2,), I32),
            pltpu.VMEM((8, 16), F32),
            pltpu.VMEM((4, 3, 1, bsz), I32),
            pltpu.VMEM((2, 1, bsz), F32),
            pltpu.SemaphoreType.DMA((4,)),
            pltpu.SemaphoreType.DMA((2,)),
            pltpu.SemaphoreType.DMA((2,)),
            pltpu.SemaphoreType.DMA((2,)),
            pltpu.SemaphoreType.DMA((2,)),
            pltpu.VMEM_SHARED((n, d), F32),
        ],
    )
    def k(feat_h, e_h, npk_h, par_h, att_h, s_h,
          rows_a, rows_b, normpkv, parv, ebuf, attb,
          isem, asem, bsem, wsem, ssem, s_sh):
        cid = lax.axis_index("c")
        sid = lax.axis_index("s")
        wid = cid * 16 + sid
        pltpu.sync_copy(npk_h, normpkv)
        pltpu.sync_copy(par_h, parv)
        rstart = sid * rows_bt
        nch = jnp.where(sid == 15, nch_last, nch_base)
        z16 = jnp.zeros((16,), F32)
        for i in range(16):
            for j in range(8):
                rows_a[0, i, pl.ds(j * 16, 16)] = z16

        def zs(i, c):
            pltpu.sync_copy(rows_a.at[0, pl.ds(0, 16)],
                            s_sh.at[pl.ds(rstart + i * 16, 16)])
            return c

        lax.fori_loop(0, nch, zs, 0)
        plsc.subcore_barrier()

        zeros_i = jnp.zeros((16,), I32)

        def idx_start(bt):
            st = bt & 3
            pltpu.async_copy(e_h.at[wid, bt], ebuf.at[st], isem.at[st])

        def idx_wait(bt):
            st = bt & 3
            pltpu.make_async_copy(
                e_h.at[wid, bt], ebuf.at[st], isem.at[st]).wait()

        def gather(bt):
            st = bt & 3
            rs = bt & 1
            pltpu.async_copy(feat_h.at[ebuf.at[st, 0, 0]], rows_a.at[rs],
                             asem.at[rs])
            pltpu.async_copy(feat_h.at[ebuf.at[st, 1, 0]], rows_b.at[rs],
                             bsem.at[rs])

        def gather_wait(bt):
            st = bt & 3
            rs = bt & 1
            pltpu.make_async_copy(
                feat_h.at[ebuf.at[st, 0, 0]], rows_a.at[rs], asem.at[rs]).wait()
            pltpu.make_async_copy(
                feat_h.at[ebuf.at[st, 1, 0]], rows_b.at[rs], bsem.at[rs]).wait()

        def group_of(bt):
            st = bt & 3
            rs = bt & 1

            def group(g, c):
                @pl.when(g == 2)
                def _():
                    @pl.when(bt >= 1)
                    def _():
                        scatter_wait(bt - 1)

                    @pl.when(bt + 1 < nb)
                    def _():
                        idx_wait(bt + 1)
                        gather(bt + 1)

                go = g * 16
                lanes = lax.iota(I32, 16)
                svec = jnp.zeros((16,), F32)
                for ee in range(16):
                    r0 = go + ee
                    va = [rows_a[rs, r0, pl.ds(ch * 16, 16)]
                          for ch in range(8)]
                    vb = [rows_b[rs, r0, pl.ds(ch * 16, 16)]
                          for ch in range(8)]
                    pr = [va[ch] * vb[ch] for ch in range(8)]
                    t0 = (pr[0] + pr[1]) + (pr[2] + pr[3])
                    t1 = (pr[4] + pr[5]) + (pr[6] + pr[7])
                    svec = jnp.where(lanes == ee, jnp.sum(t0 + t1), svec)
                ets = ebuf[st, 2, 0, pl.ds(go, 16)]
                tb = plsc.load_gather(parv, [zeros_i, ets])
                s = svec * inv_sqrt_d + tb
                prow_c = parv[1, pl.ds(0, 16)]
                prow_w = parv[2, pl.ds(0, 16)]
                prow_a = parv[3, pl.ds(0, 16)]
                prow_b = parv[4, pl.ds(0, 16)]
                msum = jnp.zeros((16,), F32)
                yacc = jnp.zeros((16,), F32)
                for r in range(4):
                    dd = s - prow_c[r]
                    m = jnp.exp(dd * dd * prow_w[r])
                    msum = msum + m
                    yacc = yacc + m * (prow_a[r] * s + prow_b[r])
                z = yacc / (msum + 1e-9)
                att = 1.0 / (1.0 + jnp.exp(-z))
                attb[rs, 0, pl.ds(go, 16)] = att
                srcs = ebuf[st, 0, 0, pl.ds(go, 16)]
                g2 = plsc.load_gather(normpkv, [lax.shift_right_logical(srcs, 1)])
                odd = srcs & 1
                bits = jnp.where(odd == 1, g2 & (-65536),
                                 lax.shift_left(g2, 16))
                nrm = plsc.bitcast(bits, F32)
                scv = att * nrm
                for ee in range(16):
                    r0 = go + ee
                    sc = scv[ee]
                    pr = [rows_a[rs, r0, pl.ds(ch * 16, 16)] * sc
                          for ch in range(8)]
                    for ch in range(8):
                        rows_b[rs, r0, pl.ds(ch * 16, 16)] = pr[ch]
                return c

            return group

        def scatter_start(bt):
            st = bt & 3
            rs = bt & 1
            pltpu.async_copy(rows_b.at[rs], s_sh.at[ebuf.at[st, 1, 0]],
                             ssem.at[rs], add=True)

        def scatter_wait(bt):
            st = bt & 3
            rs = bt & 1
            pltpu.make_async_copy(
                rows_b.at[rs], s_sh.at[ebuf.at[st, 1, 0]], ssem.at[rs]).wait()

        def att_start(bt):
            rs = bt & 1
            pltpu.async_copy(attb.at[rs], att_h.at[wid, bt], wsem.at[rs])

        def att_wait(bt):
            rs = bt & 1
            pltpu.make_async_copy(
                attb.at[rs], att_h.at[wid, bt], wsem.at[rs]).wait()

        idx_start(0)
        idx_start(1)
        idx_wait(0)
        gather(0)

        def step(t, c):
            gather_wait(t)

            @pl.when(t >= 2)
            def _():
                att_wait(t - 2)

            lax.fori_loop(0, ng, group_of(t), 0)
            scatter_start(t)
            att_start(t)

            @pl.when(t + 2 < nb)
            def _():
                idx_start(t + 2)

            return c

        lax.fori_loop(0, nb, step, 0)
        scatter_wait(nb - 1)
        att_wait(nb - 2)
        att_wait(nb - 1)
        plsc.subcore_barrier()

        def dp(i, c):
            pltpu.sync_copy(s_sh.at[pl.ds(rstart + i * 16, 16)],
                            s_h.at[cid, pl.ds(rstart + i * 16, 16)])
            return c

        lax.fori_loop(0, nch, dp, 0)

    return k(feat, e3, normpk, params)


def _sc_edge2(y, e3b):
    n, d = y.shape
    nw, nb, _, _, bsz = e3b.shape
    ng = bsz // 16
    rows_bt = ((n // 16) // 16) * 16
    nch_base = rows_bt // 16
    nch_last = (n - 15 * rows_bt) // 16
    mesh = plsc.VectorSubcoreMesh(core_axis_name="c", subcore_axis_name="s")

    @functools.partial(
        pl.kernel, mesh=mesh,
        compiler_params=pltpu.CompilerParams(needs_layout_passes=False),
        out_type=jax.ShapeDtypeStruct((2, n, d), F32),
        scratch_types=[
            pltpu.VMEM((2, bsz, d), F32),
            pltpu.VMEM((2, bsz, d), F32),
            pltpu.VMEM((4, 3, 1, bsz), I32),
            pltpu.SemaphoreType.DMA((4,)),
            pltpu.SemaphoreType.DMA((2,)),
            pltpu.SemaphoreType.DMA((2,)),
            pltpu.VMEM_SHARED((n, d), F32),
        ],
    )
    def k(y_h, e_h, s_h, rows, msgb, ebuf, isem, gsem, ssem, s_sh):
        cid = lax.axis_index("c")
        sid = lax.axis_index("s")
        wid = cid * 16 + sid
        rstart = sid * rows_bt
        nch = jnp.where(sid == 15, nch_last, nch_base)
        z16 = jnp.zeros((16,), F32)
        for i in range(16):
            for j in range(8):
                msgb[0, i, pl.ds(j * 16, 16)] = z16

        def zs(i, c):
            pltpu.sync_copy(msgb.at[0, pl.ds(0, 16)],
                            s_sh.at[pl.ds(rstart + i * 16, 16)])
            return c

        lax.fori_loop(0, nch, zs, 0)
        plsc.subcore_barrier()

        def idx_start(bt):
            st = bt & 3
            pltpu.async_copy(e_h.at[wid, bt], ebuf.at[st], isem.at[st])

        def idx_wait(bt):
            st = bt & 3
            pltpu.make_async_copy(
                e_h.at[wid, bt], ebuf.at[st], isem.at[st]).wait()

        def gather(bt):
            st = bt & 3
            rs = bt & 1
            pltpu.async_copy(y_h.at[ebuf.at[st, 0, 0]], rows.at[rs], gsem.at[rs])

        def gather_wait(bt):
            st = bt & 3
            rs = bt & 1
            pltpu.make_async_copy(
                y_h.at[ebuf.at[st, 0, 0]], rows.at[rs], gsem.at[rs]).wait()

        def group_of(bt):
            st = bt & 3
            rs = bt & 1

            def group(g, c):
                go = g * 16
                av16 = plsc.bitcast(ebuf[st, 2, 0, pl.ds(go, 16)], F32)
                for ee in range(16):
                    r0 = go + ee
                    av = av16[ee]
                    pr = [rows[rs, r0, pl.ds(ch * 16, 16)] * av
                          for ch in range(8)]
                    for ch in range(8):
                        msgb[rs, r0, pl.ds(ch * 16, 16)] = pr[ch]
                return c

            return group

        def scatter_start(bt):
            st = bt & 3
            rs = bt & 1
            pltpu.async_copy(msgb.at[rs], s_sh.at[ebuf.at[st, 1, 0]],
                             ssem.at[rs], add=True)

        def scatter_wait(bt):
            st = bt & 3
            rs = bt & 1
            pltpu.make_async_copy(
                msgb.at[rs], s_sh.at[ebuf.at[st, 1, 0]], ssem.at[rs]).wait()

        idx_start(0)
        idx_start(1)
        idx_wait(0)
        gather(0)

        def step(t, c):
            gather_wait(t)

            @pl.when(t >= 2)
            def _():
                scatter_wait(t - 2)

            @pl.when(t + 1 < nb)
            def _():
                idx_wait(t + 1)
                gather(t + 1)

            lax.fori_loop(0, ng, group_of(t), 0)
            scatter_start(t)

            @pl.when(t + 2 < nb)
            def _():
                idx_start(t + 2)

            return c

        lax.fori_loop(0, nb, step, 0)
        scatter_wait(nb - 2)
        scatter_wait(nb - 1)
        plsc.subcore_barrier()

        def dp(i, c):
            pltpu.sync_copy(s_sh.at[pl.ds(rstart + i * 16, 16)],
                            s_h.at[cid, pl.ds(rstart + i * 16, 16)])
            return c

        lax.fori_loop(0, nch, dp, 0)

    return k(y, e3b)


def _tc_layer1(s1, w, b_row, ns_col, nd_col):
    _, n, d = s1.shape
    h = w.shape[1]
    blk = 2000

    def body(s_ref, w_ref, b_ref, ns_ref, nd_ref, out_ref):
        s = s_ref[0] + s_ref[1]
        hh = jnp.dot(s, w_ref[...], preferred_element_type=F32)
        x = jnp.maximum(hh * nd_ref[...] + b_ref[...], 0.0)
        out_ref[...] = x * ns_ref[...]

    return pl.pallas_call(
        body,
        grid=(n // blk,),
        in_specs=[
            pl.BlockSpec((2, blk, d), lambda i: (0, i, 0)),
            pl.BlockSpec((d, h), lambda i: (0, 0)),
            pl.BlockSpec((1, h), lambda i: (0, 0)),
            pl.BlockSpec((blk, 1), lambda i: (i, 0)),
            pl.BlockSpec((blk, 1), lambda i: (i, 0)),
        ],
        out_specs=pl.BlockSpec((blk, h), lambda i: (i, 0)),
        out_shape=jax.ShapeDtypeStruct((n, h), F32),
    )(s1, w, b_row, ns_col, nd_col)


def _tc_final(s2, w, b_row, nd_col):
    _, n, d = s2.shape
    h = w.shape[1]
    blk = 2000
    nblk = n // blk
    neg = -3.0e38

    def body(s_ref, w_ref, b_ref, nd_ref, out_ref):
        k = pl.program_id(0)
        s = s_ref[0] + s_ref[1]
        hh = jnp.dot(s, w_ref[...], preferred_element_type=F32)
        x = jnp.maximum(hh * nd_ref[...] + b_ref[...], 0.0)
        rows = (k * blk
                + lax.broadcasted_iota(I32, (blk, h), 0))
        valid = rows >= 2
        psum = jnp.sum(jnp.where(valid, x, 0.0), axis=0, keepdims=True)
        pmax = jnp.max(jnp.where(valid, x, neg), axis=0, keepdims=True)
        pmin = jnp.min(jnp.where(valid, x, -neg), axis=0, keepdims=True)

        @pl.when(k == 0)
        def _():
            out_ref[0:1, :] = x[0:1, :]
            out_ref[1:2, :] = x[1:2, :]
            out_ref[2:3, :] = psum
            out_ref[3:4, :] = pmax
            out_ref[4:5, :] = pmin

        @pl.when(k > 0)
        def _():
            out_ref[2:3, :] = out_ref[2:3, :] + psum
            out_ref[3:4, :] = jnp.maximum(out_ref[3:4, :], pmax)
            out_ref[4:5, :] = jnp.minimum(out_ref[4:5, :], pmin)

        @pl.when(k == nblk - 1)
        def _():
            out_ref[2:3, :] = out_ref[2:3, :] * (1.0 / (n - 2))

    return pl.pallas_call(
        body,
        grid=(nblk,),
        in_specs=[
            pl.BlockSpec((2, blk, d), lambda i: (0, i, 0)),
            pl.BlockSpec((d, h), lambda i: (0, 0)),
            pl.BlockSpec((1, h), lambda i: (0, 0)),
            pl.BlockSpec((blk, 1), lambda i: (i, 0)),
        ],
        out_specs=pl.BlockSpec((5, h), lambda i: (0, 0)),
        out_shape=jax.ShapeDtypeStruct((5, h), F32),
    )(s2, w, b_row, nd_col)


def kernel(feat, edge_index, etypes, W1, b1, W2, b2,
           type_bias, centers, sigmas, a_r, b_r):
    n = feat.shape[0]
    src = edge_index[0]
    dst = edge_index[1]
    params = jnp.zeros((8, 16), F32)
    params = (params.at[0, :4].set(type_bias)
                    .at[1, :4].set(centers)
                    .at[2, :4].set(-0.5 / (sigmas * sigmas))
                    .at[3, :4].set(a_r)
                    .at[4, :4].set(b_r))
    nw, bsz = 32, 80
    nb = src.shape[0] // (nw * bsz)
    src3 = src.reshape(nw, nb, bsz)
    dst3 = dst.reshape(nw, nb, bsz)
    et3 = etypes.reshape(nw, nb, bsz)
    e3 = jnp.stack([src3, dst3, et3], axis=2).reshape(nw, nb, 3, 1, bsz)
    degpart = _sc_degrees(src, dst, n)
    norms = _tc_norms(degpart)
    ns_col = norms[0].reshape(n, 1)
    nd_col = norms[1].reshape(n, 1)
    normpk = lax.bitcast_convert_type(
        norms[0].astype(jnp.bfloat16).reshape(n // 2, 2), I32)
    att3, s1 = _sc_edge1(feat, e3, normpk, params)
    y = _tc_layer1(s1, W1, b1.reshape(1, -1), ns_col, nd_col)
    atti = lax.bitcast_convert_type(att3, I32).reshape(nw, nb, bsz)
    e3b = jnp.stack([src3, dst3, atti], axis=2).reshape(nw, nb, 3, 1, bsz)
    s2 = _sc_edge2(y, e3b)
    return _tc_final(s2, W2, b2.reshape(1, -1), nd_col)
